# baseline (speedup 1.0000x reference)
import jax
import jax.numpy as jnp
from jax import lax
from jax.experimental import pallas as pl
from jax.experimental.pallas import tpu as pltpu
from jax.experimental.pallas import tpu_sc as plsc

B = 128
G = 2000
P = G + 1
PPAD = 2016
D = 512
E_ROWS = 53
CLS_ROW = 60697
NC = 2
NS = 16
NW = NC * NS
RPW = B // NW
C = 32
NCH = PPAD // C
TAIL = P - (NCH - 1) * C
LANES = 16


def _body(eidx_hbm, gtab_hbm, etab_hbm, out_hbm,
          idxall, etab_v, eb0, eb1, eb2, eb3, tailbuf, erow_s,
          sg0, sg1, sg2, sg3, so0, so1, so2, so3):
    cid = lax.axis_index("c")
    sid = lax.axis_index("s")
    wid = sid * NC + cid
    ebuf = (eb0, eb1, eb2, eb3)
    sem_gene = (sg0, sg1, sg2, sg3)
    sem_out = (so0, so1, so2, so3)

    def issue_gene(k, q):
        pltpu.async_copy(gtab_hbm.at[pl.ds(k * C, C), :], ebuf[q],
                         sem_gene[q])

    def wait_gene(q):
        pltpu.make_async_copy(gtab_hbm.at[pl.ds(0, C), :], ebuf[q],
                              sem_gene[q]).wait()

    def issue_write(k, r, q):
        b = wid * RPW + r
        pltpu.async_copy(ebuf[q], out_hbm.at[b, pl.ds(k * C, C), :],
                         sem_out[q])

    def wait_write(q):
        pltpu.make_async_copy(ebuf[q], out_hbm.at[0, pl.ds(0, C), :],
                              sem_out[q]).wait()

    def accumulate(k, r, q):
        idx_base = r * PPAD + k * C
        for g in range(C // LANES):
            ev = idxall[pl.ds(idx_base + g * LANES, LANES)]
            for tt in range(LANES):
                erow_s[g * LANES + tt] = ev[tt]

        def per_tok(t2, carry):
            t = t2 * 2
            e0 = erow_s[t]
            e1 = erow_s[t + 1]
            for c in range(D // LANES):
                sl = pl.ds(c * LANES, LANES)
                plsc.addupdate(ebuf[q].at[t, sl], etab_v[e0, sl])
                plsc.addupdate(ebuf[q].at[t + 1, sl], etab_v[e1, sl])
            return carry

        lax.fori_loop(0, C // 2, per_tok, 0)

    pltpu.sync_copy(eidx_hbm.at[pl.ds(wid * RPW * PPAD, RPW * PPAD)], idxall)
    pltpu.sync_copy(etab_hbm, etab_v)
    issue_gene(0, 0)
    issue_gene(0, 1)

    for r in range(4):
        if r >= 2:
            wait_write((r + 2) % 4)
        issue_gene(0 if r < 2 else 1, (r + 2) % 4)
        wait_gene(r)
        accumulate(0, r, r)
        issue_write(0, r, r)

    def steady(k, carry):
        for r in range(4):
            wait_write((r + 2) % 4)
            issue_gene(k if r < 2 else k + 1, (r + 2) % 4)
            wait_gene(r)
            accumulate(k, r, r)
            issue_write(k, r, r)
        return carry

    lax.fori_loop(1, NCH - 1, steady, 0)

    for r in range(4):
        if r < 2:
            wait_write(r + 2)
            issue_gene(NCH - 1, r + 2)
        wait_gene(r)
        accumulate(NCH - 1, r, r)

        def tail_tok(t, carry):
            for c in range(D // LANES):
                sl = pl.ds(c * LANES, LANES)
                tailbuf[t, sl] = ebuf[r][t, sl]
            return carry

        lax.fori_loop(0, TAIL, tail_tok, 0)
        b = wid * RPW + r
        pltpu.sync_copy(tailbuf, out_hbm.at[b, pl.ds((NCH - 1) * C, TAIL), :])


def kernel(expression, gene_table, expr_table):
    expr_i = expression.astype(jnp.int32)
    gtab = jnp.concatenate(
        [gene_table[CLS_ROW:CLS_ROW + 1], gene_table[:G],
         jnp.zeros((PPAD - P, D), jnp.float32)], axis=0)
    etab2 = jnp.concatenate(
        [expr_table, jnp.zeros((1, D), jnp.float32)], axis=0)
    zero_col = jnp.full((B, 1), E_ROWS - 1, jnp.int32)
    pad_cols = jnp.full((B, PPAD - P), E_ROWS - 1, jnp.int32)
    eidx = jnp.concatenate([zero_col, expr_i, pad_cols], axis=1)
    eidx_flat = eidx.reshape(B * PPAD)

    mesh = plsc.VectorSubcoreMesh(core_axis_name="c", subcore_axis_name="s")
    emb = pl.kernel(
        _body,
        mesh=mesh,
        out_type=jax.ShapeDtypeStruct((B, P, D), jnp.float32),
        scratch_types=[
            pltpu.VMEM((RPW * PPAD,), jnp.int32),
            pltpu.VMEM((E_ROWS, D), jnp.float32),
            pltpu.VMEM((C, D), jnp.float32),
            pltpu.VMEM((C, D), jnp.float32),
            pltpu.VMEM((C, D), jnp.float32),
            pltpu.VMEM((C, D), jnp.float32),
            pltpu.VMEM((TAIL, D), jnp.float32),
            pltpu.SMEM((C,), jnp.int32),
            pltpu.SemaphoreType.DMA,
            pltpu.SemaphoreType.DMA,
            pltpu.SemaphoreType.DMA,
            pltpu.SemaphoreType.DMA,
            pltpu.SemaphoreType.DMA,
            pltpu.SemaphoreType.DMA,
            pltpu.SemaphoreType.DMA,
            pltpu.SemaphoreType.DMA,
        ],
    )(eidx_flat, gtab, etab2)
    mask = jnp.ones((B, P), dtype=jnp.float32)
    return emb, mask


# speedup vs baseline: 1.0072x; 1.0072x over previous
import jax
import jax.numpy as jnp
from jax import lax
from jax.experimental import pallas as pl
from jax.experimental.pallas import tpu as pltpu
from jax.experimental.pallas import tpu_sc as plsc

B = 128
G = 2000
P = G + 1
PPAD = 2016
D = 512
E_ROWS = 53
CLS_ROW = 60697
NC = 2
NS = 16
NW = NC * NS
RPW = B // NW
C = 32
NCH = PPAD // C
TAIL = P - (NCH - 1) * C
LANES = 16


def _body(eidx_hbm, gtab_hbm, etab_hbm, out_hbm,
          idxall, etab_v, eb0, eb1, eb2, eb3, tailbuf, erow_s,
          sg0, sg1, sg2, sg3, so0, so1, so2, so3):
    cid = lax.axis_index("c")
    sid = lax.axis_index("s")
    wid = sid * NC + cid
    ebuf = (eb0, eb1, eb2, eb3)
    sem_gene = (sg0, sg1, sg2, sg3)
    sem_out = (so0, so1, so2, so3)

    def issue_gene(k, q):
        pltpu.async_copy(gtab_hbm.at[pl.ds(k * C, C), :], ebuf[q],
                         sem_gene[q])

    def wait_gene(q):
        pltpu.make_async_copy(gtab_hbm.at[pl.ds(0, C), :], ebuf[q],
                              sem_gene[q]).wait()

    def issue_write(k, r, q):
        b = wid * RPW + r
        pltpu.async_copy(ebuf[q], out_hbm.at[b, pl.ds(k * C, C), :],
                         sem_out[q])

    def wait_write(q):
        pltpu.make_async_copy(ebuf[q], out_hbm.at[0, pl.ds(0, C), :],
                              sem_out[q]).wait()

    def accumulate(k, r, q):
        idx_base = r * PPAD + k * C
        for g in range(C // LANES):
            ev = idxall[pl.ds(idx_base + g * LANES, LANES)]
            for tt in range(LANES):
                erow_s[g * LANES + tt] = ev[tt]

        def per_tok(t, carry):
            e_row = erow_s[t]
            for c in range(D // LANES):
                sl = pl.ds(c * LANES, LANES)
                plsc.addupdate(ebuf[q].at[t, sl], etab_v[e_row, sl])
            return carry

        lax.fori_loop(0, C, per_tok, 0)

    pltpu.sync_copy(eidx_hbm.at[pl.ds(wid * RPW * PPAD, RPW * PPAD)], idxall)
    pltpu.sync_copy(etab_hbm, etab_v)
    issue_gene(0, 0)
    issue_gene(0, 1)

    for r in range(4):
        if r >= 2:
            wait_write((r + 2) % 4)
        issue_gene(0 if r < 2 else 1, (r + 2) % 4)
        wait_gene(r)
        accumulate(0, r, r)
        issue_write(0, r, r)

    def steady(k, carry):
        for r in range(4):
            wait_write((r + 2) % 4)
            issue_gene(k if r < 2 else k + 1, (r + 2) % 4)
            wait_gene(r)
            accumulate(k, r, r)
            issue_write(k, r, r)
        return carry

    lax.fori_loop(1, NCH - 1, steady, 0)

    for r in range(4):
        if r < 2:
            wait_write(r + 2)
            issue_gene(NCH - 1, r + 2)
        wait_gene(r)
        accumulate(NCH - 1, r, r)

        def tail_tok(t, carry):
            for c in range(D // LANES):
                sl = pl.ds(c * LANES, LANES)
                tailbuf[t, sl] = ebuf[r][t, sl]
            return carry

        lax.fori_loop(0, TAIL, tail_tok, 0)
        b = wid * RPW + r
        pltpu.sync_copy(tailbuf, out_hbm.at[b, pl.ds((NCH - 1) * C, TAIL), :])


def kernel(expression, gene_table, expr_table):
    expr_i = expression.astype(jnp.int32)
    gtab = jnp.concatenate(
        [gene_table[CLS_ROW:CLS_ROW + 1], gene_table[:G],
         jnp.zeros((PPAD - P, D), jnp.float32)], axis=0)
    etab2 = jnp.concatenate(
        [expr_table, jnp.zeros((1, D), jnp.float32)], axis=0)
    zero_col = jnp.full((B, 1), E_ROWS - 1, jnp.int32)
    pad_cols = jnp.full((B, PPAD - P), E_ROWS - 1, jnp.int32)
    eidx = jnp.concatenate([zero_col, expr_i, pad_cols], axis=1)
    eidx_flat = eidx.reshape(B * PPAD)

    mesh = plsc.VectorSubcoreMesh(core_axis_name="c", subcore_axis_name="s")
    emb = pl.kernel(
        _body,
        mesh=mesh,
        out_type=jax.ShapeDtypeStruct((B, P, D), jnp.float32),
        scratch_types=[
            pltpu.VMEM((RPW * PPAD,), jnp.int32),
            pltpu.VMEM((E_ROWS, D), jnp.float32),
            pltpu.VMEM((C, D), jnp.float32),
            pltpu.VMEM((C, D), jnp.float32),
            pltpu.VMEM((C, D), jnp.float32),
            pltpu.VMEM((C, D), jnp.float32),
            pltpu.VMEM((TAIL, D), jnp.float32),
            pltpu.SMEM((C,), jnp.int32),
            pltpu.SemaphoreType.DMA,
            pltpu.SemaphoreType.DMA,
            pltpu.SemaphoreType.DMA,
            pltpu.SemaphoreType.DMA,
            pltpu.SemaphoreType.DMA,
            pltpu.SemaphoreType.DMA,
            pltpu.SemaphoreType.DMA,
            pltpu.SemaphoreType.DMA,
        ],
    )(eidx_flat, gtab, etab2)
    mask = jnp.ones((B, P), dtype=jnp.float32)
    return emb, mask
